# baseline (speedup 1.0000x reference)
import functools

import jax
import jax.numpy as jnp
from jax import lax
from jax.experimental import pallas as pl
from jax.experimental.pallas import tpu as pltpu
from jax.experimental.pallas import tpu_sc as plsc

BATCH = 16384
D = 64
NC = 2
NS = 16
NW = NC * NS
BPW = BATCH // NW
CHUNK = 128
NCHUNK = BPW // CHUNK

_MESH = plsc.VectorSubcoreMesh(core_axis_name="c", subcore_axis_name="s")


@functools.partial(
    pl.kernel,
    mesh=_MESH,
    compiler_params=pltpu.CompilerParams(use_tc_tiling_on_sc=False),
    out_type=(
        jax.ShapeDtypeStruct((BATCH, D), jnp.float32),
        jax.ShapeDtypeStruct((BATCH, D), jnp.float32),
    ),
    scratch_types=[
        pltpu.VMEM((BPW,), jnp.int32),
        pltpu.VMEM((BPW, D), jnp.float32),
        pltpu.VMEM((BPW,), jnp.int32),
        pltpu.VMEM((BPW, D), jnp.float32),
        pltpu.SemaphoreType.DMA,
    ],
)
def _sc_gather(user_table, movie_table, user_ids, movie_ids,
               user_out, movie_gath, uidx, urows, midx, mrows, sem):
    wid = lax.axis_index("s") * NC + lax.axis_index("c")
    base = wid * BPW
    pltpu.sync_copy(user_ids.at[pl.ds(base, BPW)], uidx)
    pltpu.sync_copy(movie_ids.at[pl.ds(base, BPW)], midx)
    copies = []
    for j in range(NCHUNK):
        sl = pl.ds(j * CHUNK, CHUNK)
        copies.append(pltpu.async_copy(
            user_table.at[uidx.at[sl]], urows.at[sl], sem))
        copies.append(pltpu.async_copy(
            movie_table.at[midx.at[sl]], mrows.at[sl], sem))
    for c in copies:
        c.wait()
    pltpu.sync_copy(urows, user_out.at[pl.ds(base, BPW)])
    pltpu.sync_copy(mrows, movie_gath.at[pl.ds(base, BPW)])


def _tc_body(x_ref, w_ref, b_ref, g_ref, o_ref):
    o_ref[...] = (
        jnp.dot(x_ref[...], w_ref[...], preferred_element_type=jnp.float32)
        + b_ref[...] + g_ref[...]
    )


def kernel(movie_x, user_table, movie_table, W, b, user_node_id, movie_node_id):
    user_out, movie_gath = _sc_gather(
        user_table, movie_table, user_node_id, movie_node_id)
    BM = 2048
    movie_out = pl.pallas_call(
        _tc_body,
        grid=(BATCH // BM,),
        in_specs=[
            pl.BlockSpec((BM, 20), lambda i: (i, 0)),
            pl.BlockSpec((20, D), lambda i: (0, 0)),
            pl.BlockSpec((1, D), lambda i: (0, 0)),
            pl.BlockSpec((BM, D), lambda i: (i, 0)),
        ],
        out_specs=pl.BlockSpec((BM, D), lambda i: (i, 0)),
        out_shape=jax.ShapeDtypeStruct((BATCH, D), jnp.float32),
    )(movie_x, W, b.reshape(1, D), movie_gath)
    return (user_out, movie_out)
